# baseline (speedup 1.0000x reference)
import functools

import jax
import jax.numpy as jnp
from jax import lax
from jax.experimental import pallas as pl
from jax.experimental.pallas import tpu as pltpu
from jax.experimental.pallas import tpu_sc as plsc

N_NODES = 10000
N_EDGES = 320000
NFEAT = 128

NC = 2
NS = 16
LANES = 16

CHUNK = 128
N_PAD = 10240
CHUNKS_PER_TILE = 80
EDGES_PER_TILE = CHUNKS_PER_TILE * CHUNK
E_PAD = EDGES_PER_TILE * NC * NS
ROWS_PER_TILE = N_PAD // NS
NBUF = 2

@functools.lru_cache(maxsize=None)
def _mesh():
  return plsc.VectorSubcoreMesh(
      core_axis_name="c", subcore_axis_name="s", num_cores=NC, num_subcores=NS
  )


def _zero_vmem_2d(ref, n_rows, n_cols):
  z = jnp.zeros((LANES,), jnp.float32)

  def body(i, _):
    for j in range(n_cols // LANES):
      ref[i, pl.ds(j * LANES, LANES)] = z
    return 0

  lax.fori_loop(0, n_rows, body, 0)


def _zero_vmem_1d(ref, n):
  z = jnp.zeros((LANES,), jnp.float32)
  for j in range(n // LANES):
    ref[pl.ds(j * LANES, LANES)] = z


def _deg_body(src_hbm, dst_hbm, out_hbm, sidx_v, didx_v, ones_v, zbuf_v,
              sem_a, sem_b, acc_out, acc_in):
  cid = lax.axis_index("c")
  sid = lax.axis_index("s")

  for j in range(CHUNK // LANES):
    ones_v[pl.ds(j * LANES, LANES)] = jnp.ones((LANES,), jnp.float32)
  _zero_vmem_1d(zbuf_v, ROWS_PER_TILE)

  pltpu.sync_copy(zbuf_v, acc_out.at[pl.ds(sid * ROWS_PER_TILE,
                                           ROWS_PER_TILE)])
  pltpu.sync_copy(zbuf_v, acc_in.at[pl.ds(sid * ROWS_PER_TILE,
                                          ROWS_PER_TILE)])
  pltpu.sync_copy(src_hbm.at[cid, sid], sidx_v)
  pltpu.sync_copy(dst_hbm.at[cid, sid], didx_v)
  plsc.subcore_barrier()

  def chunk_body(k, _):
    pltpu.async_copy(ones_v, acc_out.at[sidx_v.at[k]], sem_a, add=True)
    pltpu.async_copy(ones_v, acc_in.at[didx_v.at[k]], sem_b, add=True)
    return 0

  lax.fori_loop(0, CHUNKS_PER_TILE, chunk_body, 0)

  def drain_body(k, _):
    pltpu.make_async_copy(ones_v, acc_out.at[sidx_v.at[0]], sem_a).wait()
    pltpu.make_async_copy(ones_v, acc_in.at[didx_v.at[0]], sem_b).wait()
    return 0

  lax.fori_loop(0, CHUNKS_PER_TILE, drain_body, 0)
  plsc.subcore_barrier()

  row = pl.ds(sid * ROWS_PER_TILE, ROWS_PER_TILE)
  pltpu.sync_copy(acc_out.at[row], out_hbm.at[cid, 0, row])
  pltpu.sync_copy(acc_in.at[row], out_hbm.at[cid, 1, row])


@functools.lru_cache(maxsize=None)
def _deg_call():
  return pl.kernel(
      _deg_body,
      out_type=jax.ShapeDtypeStruct((NC, 2, N_PAD), jnp.float32),
      mesh=_mesh(),
      scratch_types=[
          pltpu.VMEM((CHUNKS_PER_TILE, CHUNK), jnp.int32),
          pltpu.VMEM((CHUNKS_PER_TILE, CHUNK), jnp.int32),
          pltpu.VMEM((CHUNK,), jnp.float32),
          pltpu.VMEM((ROWS_PER_TILE,), jnp.float32),
          pltpu.SemaphoreType.DMA,
          pltpu.SemaphoreType.DMA,
          pltpu.VMEM_SHARED((N_PAD,), jnp.float32),
          pltpu.VMEM_SHARED((N_PAD,), jnp.float32),
      ],
  )


PHASES = 2
PHASE_CHUNKS = CHUNKS_PER_TILE // PHASES


def _agg_body(rows_hbm, src_hbm, dst_hbm, out_hbm, sidx_v, didx_v, bufs_v,
              semg0, semg1, acc):
  cid = lax.axis_index("c")
  sid = lax.axis_index("s")
  semg = (semg0, semg1)
  bufs = [bufs_v.at[b] for b in range(NBUF)]

  _zero_vmem_2d(bufs_v.at[0], CHUNK, NFEAT)
  for m in range(ROWS_PER_TILE // CHUNK):
    pltpu.sync_copy(
        bufs[0], acc.at[pl.ds(sid * ROWS_PER_TILE + m * CHUNK, CHUNK)])
  plsc.subcore_barrier()

  def gather(k, b):
    pltpu.async_copy(rows_hbm.at[sidx_v.at[k]], bufs[b], semg[b])

  def gather_wait(k, b):
    pltpu.make_async_copy(rows_hbm.at[sidx_v.at[k]], bufs[b],
                          semg[b]).wait()

  def scatter(k, b):
    pltpu.sync_copy(bufs[b], acc.at[didx_v.at[k]], add=True)

  for phase in range(PHASES):
    base = phase * PHASE_CHUNKS
    pltpu.sync_copy(src_hbm.at[cid, sid, pl.ds(base, PHASE_CHUNKS)], sidx_v)
    pltpu.sync_copy(dst_hbm.at[cid, sid, pl.ds(base, PHASE_CHUNKS)], didx_v)

    for b in range(NBUF):
      gather(b, b)

    def round_body(i, _):
      for b in range(NBUF):
        k = i * NBUF + b
        gather_wait(k, b)
        scatter(k, b)
        gather(k + NBUF, b)
      return 0

    lax.fori_loop(0, PHASE_CHUNKS // NBUF - 1, round_body, 0)
    for b in range(NBUF):
      k = PHASE_CHUNKS - NBUF + b
      gather_wait(k, b)
      scatter(k, b)

  plsc.subcore_barrier()
  row = pl.ds(sid * ROWS_PER_TILE, ROWS_PER_TILE)
  pltpu.sync_copy(acc.at[row], out_hbm.at[cid, row])


@functools.lru_cache(maxsize=None)
def _agg_call():
  return pl.kernel(
      _agg_body,
      out_type=jax.ShapeDtypeStruct((NC, N_PAD, NFEAT), jnp.float32),
      mesh=_mesh(),
      scratch_types=[
          pltpu.VMEM((PHASE_CHUNKS, CHUNK), jnp.int32),
          pltpu.VMEM((PHASE_CHUNKS, CHUNK), jnp.int32),
          pltpu.VMEM((NBUF, CHUNK, NFEAT), jnp.float32),
          pltpu.SemaphoreType.DMA,
          pltpu.SemaphoreType.DMA,
          pltpu.VMEM_SHARED((N_PAD, NFEAT), jnp.float32),
      ],
  )


ROW_BLK = 1000
GRID = N_NODES // ROW_BLK


def _norm_from(deg_blk):
  return lax.rsqrt(jnp.maximum(deg_blk, 1.0))


def _norms_body(degp_ref, ns_ref, nd_ref):
  ns_ref[...] = _norm_from(degp_ref[0, 0, :] + degp_ref[1, 0, :])[:, None]
  nd_ref[...] = _norm_from(degp_ref[0, 1, :] + degp_ref[1, 1, :])[:, None]


def _norms(degp):
  return pl.pallas_call(
      _norms_body,
      out_shape=[
          jax.ShapeDtypeStruct((N_PAD, 1), jnp.float32),
          jax.ShapeDtypeStruct((N_PAD, 1), jnp.float32),
      ],
  )(degp)


def _mm1_body(x_ref, w_ref, ns_ref, o_ref):
  xs = x_ref[...] * ns_ref[...]
  o_ref[...] = jnp.dot(xs, w_ref[...], preferred_element_type=jnp.float32)


def _mm1(x, w, ns):
  return pl.pallas_call(
      _mm1_body,
      grid=(GRID,),
      in_specs=[
          pl.BlockSpec((ROW_BLK, NFEAT), lambda i: (i, 0)),
          pl.BlockSpec((NFEAT, NFEAT), lambda i: (0, 0)),
          pl.BlockSpec((ROW_BLK, 1), lambda i: (i, 0)),
      ],
      out_specs=pl.BlockSpec((ROW_BLK, NFEAT), lambda i: (i, 0)),
      out_shape=jax.ShapeDtypeStruct((N_PAD, NFEAT), jnp.float32),
  )(x, w, ns)


def _mid_body(aggp_ref, ns_ref, nd_ref, b_ref, w_ref, o_ref):
  agg = aggp_ref[0] + aggp_ref[1]
  h = jax.nn.relu(agg * nd_ref[...] + b_ref[...][None, :])
  hs = h * ns_ref[...]
  o_ref[...] = jnp.dot(hs, w_ref[...], preferred_element_type=jnp.float32)


def _mid(aggp, ns, nd, b, w):
  return pl.pallas_call(
      _mid_body,
      grid=(GRID,),
      in_specs=[
          pl.BlockSpec((NC, ROW_BLK, NFEAT), lambda i: (0, i, 0)),
          pl.BlockSpec((ROW_BLK, 1), lambda i: (i, 0)),
          pl.BlockSpec((ROW_BLK, 1), lambda i: (i, 0)),
          pl.BlockSpec((NFEAT,), lambda i: (0,)),
          pl.BlockSpec((NFEAT, NFEAT), lambda i: (0, 0)),
      ],
      out_specs=pl.BlockSpec((ROW_BLK, NFEAT), lambda i: (i, 0)),
      out_shape=jax.ShapeDtypeStruct((N_PAD, NFEAT), jnp.float32),
  )(aggp, ns, nd, b, w)


def _final_body(aggp_ref, nd_ref, b_ref, o_ref):
  agg = aggp_ref[0] + aggp_ref[1]
  o_ref[...] = agg * nd_ref[...] + b_ref[...][None, :]


def _final(aggp, nd, b):
  return pl.pallas_call(
      _final_body,
      grid=(GRID,),
      in_specs=[
          pl.BlockSpec((NC, ROW_BLK, NFEAT), lambda i: (0, i, 0)),
          pl.BlockSpec((ROW_BLK, 1), lambda i: (i, 0)),
          pl.BlockSpec((NFEAT,), lambda i: (0,)),
      ],
      out_specs=pl.BlockSpec((ROW_BLK, NFEAT), lambda i: (i, 0)),
      out_shape=jax.ShapeDtypeStruct((N_NODES, NFEAT), jnp.float32),
  )(aggp, nd, b)


def kernel(x, W1, b1, W2, b2, edge_index):
  ei = edge_index.astype(jnp.int32)
  real_per_tile = N_EDGES // (NC * NS)
  pad_per_tile = EDGES_PER_TILE - real_per_tile
  pad_block = jnp.broadcast_to(
      (N_NODES + jnp.arange(pad_per_tile, dtype=jnp.int32))[None, :],
      (NC * NS, pad_per_tile))

  def _pad_edges(v):
    v2 = v.reshape(NC * NS, real_per_tile)
    return jnp.concatenate([v2, pad_block], axis=1).reshape(
        NC, NS, CHUNKS_PER_TILE, CHUNK)

  src = _pad_edges(ei[0])
  dst = _pad_edges(ei[1])

  degp = _deg_call()(src, dst)
  ns, nd = _norms(degp)
  hs1 = _mm1(x, W1, ns)
  aggp1 = _agg_call()(hs1, src, dst)
  hs2 = _mid(aggp1, ns, nd, b1, W2)
  aggp2 = _agg_call()(hs2, src, dst)
  return _final(aggp2, nd, b2)


# speedup vs baseline: 1.0496x; 1.0496x over previous
import functools

import jax
import jax.numpy as jnp
from jax import lax
from jax.experimental import pallas as pl
from jax.experimental.pallas import tpu as pltpu
from jax.experimental.pallas import tpu_sc as plsc

N_NODES = 10000
N_EDGES = 320000
NFEAT = 128

NC = 2
NS = 16
LANES = 16

CHUNK = 128
N_PAD = 10240
CHUNKS_PER_TILE = 80
EDGES_PER_TILE = CHUNKS_PER_TILE * CHUNK
E_PAD = EDGES_PER_TILE * NC * NS
ROWS_PER_TILE = N_PAD // NS
NBUF = 2

@functools.lru_cache(maxsize=None)
def _mesh():
  return plsc.VectorSubcoreMesh(
      core_axis_name="c", subcore_axis_name="s", num_cores=NC, num_subcores=NS
  )


def _zero_vmem_2d(ref, n_rows, n_cols):
  z = jnp.zeros((LANES,), jnp.float32)

  def body(i, _):
    for j in range(n_cols // LANES):
      ref[i, pl.ds(j * LANES, LANES)] = z
    return 0

  lax.fori_loop(0, n_rows, body, 0)


def _zero_vmem_1d(ref, n):
  z = jnp.zeros((LANES,), jnp.float32)
  for j in range(n // LANES):
    ref[pl.ds(j * LANES, LANES)] = z


def _deg_body(src_hbm, dst_hbm, out_hbm, sidx_v, didx_v, ones_v, zbuf_v,
              sem_a, sem_b, acc_out, acc_in):
  cid = lax.axis_index("c")
  sid = lax.axis_index("s")

  for j in range(CHUNK // LANES):
    ones_v[pl.ds(j * LANES, LANES)] = jnp.ones((LANES,), jnp.float32)
  _zero_vmem_1d(zbuf_v, ROWS_PER_TILE)

  pltpu.sync_copy(zbuf_v, acc_out.at[pl.ds(sid * ROWS_PER_TILE,
                                           ROWS_PER_TILE)])
  pltpu.sync_copy(zbuf_v, acc_in.at[pl.ds(sid * ROWS_PER_TILE,
                                          ROWS_PER_TILE)])
  pltpu.sync_copy(src_hbm.at[cid, sid], sidx_v)
  pltpu.sync_copy(dst_hbm.at[cid, sid], didx_v)
  plsc.subcore_barrier()

  def chunk_body(k, _):
    pltpu.async_copy(ones_v, acc_out.at[sidx_v.at[k]], sem_a, add=True)
    pltpu.async_copy(ones_v, acc_in.at[didx_v.at[k]], sem_b, add=True)
    return 0

  lax.fori_loop(0, CHUNKS_PER_TILE, chunk_body, 0)

  def drain_body(k, _):
    pltpu.make_async_copy(ones_v, acc_out.at[sidx_v.at[0]], sem_a).wait()
    pltpu.make_async_copy(ones_v, acc_in.at[didx_v.at[0]], sem_b).wait()
    return 0

  lax.fori_loop(0, CHUNKS_PER_TILE, drain_body, 0)
  plsc.subcore_barrier()

  row = pl.ds(sid * ROWS_PER_TILE, ROWS_PER_TILE)
  pltpu.sync_copy(acc_out.at[row], out_hbm.at[cid, 0, row])
  pltpu.sync_copy(acc_in.at[row], out_hbm.at[cid, 1, row])


@functools.lru_cache(maxsize=None)
def _deg_call():
  return pl.kernel(
      _deg_body,
      out_type=jax.ShapeDtypeStruct((NC, 2, N_PAD), jnp.float32),
      mesh=_mesh(),
      scratch_types=[
          pltpu.VMEM((CHUNKS_PER_TILE, CHUNK), jnp.int32),
          pltpu.VMEM((CHUNKS_PER_TILE, CHUNK), jnp.int32),
          pltpu.VMEM((CHUNK,), jnp.float32),
          pltpu.VMEM((ROWS_PER_TILE,), jnp.float32),
          pltpu.SemaphoreType.DMA,
          pltpu.SemaphoreType.DMA,
          pltpu.VMEM_SHARED((N_PAD,), jnp.float32),
          pltpu.VMEM_SHARED((N_PAD,), jnp.float32),
      ],
  )


PHASES = 2
PHASE_CHUNKS = CHUNKS_PER_TILE // PHASES


def _agg_body(rows_hbm, src_hbm, dst_hbm, out_hbm, sidx_v, didx_v, bufs_v,
              semg0, semg1, acc):
  cid = lax.axis_index("c")
  sid = lax.axis_index("s")
  semg = (semg0, semg1)
  bufs = [bufs_v.at[b] for b in range(NBUF)]

  _zero_vmem_2d(bufs_v.at[0], CHUNK, NFEAT)
  for m in range(ROWS_PER_TILE // CHUNK):
    pltpu.sync_copy(
        bufs[0], acc.at[pl.ds(sid * ROWS_PER_TILE + m * CHUNK, CHUNK)])
  plsc.subcore_barrier()

  def gather(k, b):
    pltpu.async_copy(rows_hbm.at[sidx_v.at[k]], bufs[b], semg[b])

  def gather_wait(k, b):
    pltpu.make_async_copy(rows_hbm.at[sidx_v.at[k]], bufs[b],
                          semg[b]).wait()

  def scatter(k, b):
    pltpu.sync_copy(bufs[b], acc.at[didx_v.at[k]], add=True)

  for phase in range(PHASES):
    base = phase * PHASE_CHUNKS
    pltpu.sync_copy(src_hbm.at[cid, sid, pl.ds(base, PHASE_CHUNKS)], sidx_v)
    pltpu.sync_copy(dst_hbm.at[cid, sid, pl.ds(base, PHASE_CHUNKS)], didx_v)

    for b in range(NBUF):
      gather(b, b)

    def round_body(i, _):
      for b in range(NBUF):
        k = i * NBUF + b
        gather_wait(k, b)
        scatter(k, b)
        gather(k + NBUF, b)
      return 0

    lax.fori_loop(0, PHASE_CHUNKS // NBUF - 1, round_body, 0)
    for b in range(NBUF):
      k = PHASE_CHUNKS - NBUF + b
      gather_wait(k, b)
      scatter(k, b)

  plsc.subcore_barrier()
  row = pl.ds(sid * ROWS_PER_TILE, ROWS_PER_TILE)
  pltpu.sync_copy(acc.at[row], out_hbm.at[cid, row])


@functools.lru_cache(maxsize=None)
def _agg_call():
  return pl.kernel(
      _agg_body,
      out_type=jax.ShapeDtypeStruct((NC, N_PAD, NFEAT), jnp.float32),
      mesh=_mesh(),
      scratch_types=[
          pltpu.VMEM((PHASE_CHUNKS, CHUNK), jnp.int32),
          pltpu.VMEM((PHASE_CHUNKS, CHUNK), jnp.int32),
          pltpu.VMEM((NBUF, CHUNK, NFEAT), jnp.float32),
          pltpu.SemaphoreType.DMA,
          pltpu.SemaphoreType.DMA,
          pltpu.VMEM_SHARED((N_PAD, NFEAT), jnp.float32),
      ],
  )


ROW_BLK = 1280
GRID = N_PAD // ROW_BLK


def _norm_from(deg_blk):
  return lax.rsqrt(jnp.maximum(deg_blk, 1.0))


def _mm1_body(x_ref, w_ref, degp_ref, o_ref):
  deg_out = degp_ref[0, 0, :] + degp_ref[1, 0, :]
  xs = x_ref[...] * _norm_from(deg_out)[:, None]
  o_ref[...] = jnp.dot(xs, w_ref[...], preferred_element_type=jnp.float32)


def _mm1(x, w, degp):
  return pl.pallas_call(
      _mm1_body,
      grid=(GRID,),
      in_specs=[
          pl.BlockSpec((ROW_BLK, NFEAT), lambda i: (i, 0)),
          pl.BlockSpec((NFEAT, NFEAT), lambda i: (0, 0)),
          pl.BlockSpec((NC, 2, ROW_BLK), lambda i: (0, 0, i)),
      ],
      out_specs=pl.BlockSpec((ROW_BLK, NFEAT), lambda i: (i, 0)),
      out_shape=jax.ShapeDtypeStruct((N_PAD, NFEAT), jnp.float32),
  )(x, w, degp)


def _mid_body(aggp_ref, degp_ref, b_ref, w_ref, o_ref):
  agg = aggp_ref[0] + aggp_ref[1]
  norm_dst = _norm_from(degp_ref[0, 1, :] + degp_ref[1, 1, :])
  norm_src = _norm_from(degp_ref[0, 0, :] + degp_ref[1, 0, :])
  h = jax.nn.relu(agg * norm_dst[:, None] + b_ref[...][None, :])
  hs = h * norm_src[:, None]
  o_ref[...] = jnp.dot(hs, w_ref[...], preferred_element_type=jnp.float32)


def _mid(aggp, degp, b, w):
  return pl.pallas_call(
      _mid_body,
      grid=(GRID,),
      in_specs=[
          pl.BlockSpec((NC, ROW_BLK, NFEAT), lambda i: (0, i, 0)),
          pl.BlockSpec((NC, 2, ROW_BLK), lambda i: (0, 0, i)),
          pl.BlockSpec((NFEAT,), lambda i: (0,)),
          pl.BlockSpec((NFEAT, NFEAT), lambda i: (0, 0)),
      ],
      out_specs=pl.BlockSpec((ROW_BLK, NFEAT), lambda i: (i, 0)),
      out_shape=jax.ShapeDtypeStruct((N_PAD, NFEAT), jnp.float32),
  )(aggp, degp, b, w)


def _final_body(aggp_ref, degp_ref, b_ref, o_ref):
  agg = aggp_ref[0] + aggp_ref[1]
  norm_dst = _norm_from(degp_ref[0, 1, :] + degp_ref[1, 1, :])
  o_ref[...] = agg * norm_dst[:, None] + b_ref[...][None, :]


def _final(aggp, degp, b):
  return pl.pallas_call(
      _final_body,
      grid=(GRID,),
      in_specs=[
          pl.BlockSpec((NC, ROW_BLK, NFEAT), lambda i: (0, i, 0)),
          pl.BlockSpec((NC, 2, ROW_BLK), lambda i: (0, 0, i)),
          pl.BlockSpec((NFEAT,), lambda i: (0,)),
      ],
      out_specs=pl.BlockSpec((ROW_BLK, NFEAT), lambda i: (i, 0)),
      out_shape=jax.ShapeDtypeStruct((N_NODES, NFEAT), jnp.float32),
  )(aggp, degp, b)


def kernel(x, W1, b1, W2, b2, edge_index):
  ei = edge_index.astype(jnp.int32)
  real_per_tile = N_EDGES // (NC * NS)
  pad_per_tile = EDGES_PER_TILE - real_per_tile
  pad_block = jnp.broadcast_to(
      (N_NODES + jnp.arange(pad_per_tile, dtype=jnp.int32))[None, :],
      (NC * NS, pad_per_tile))

  def _pad_edges(v):
    v2 = v.reshape(NC * NS, real_per_tile)
    return jnp.concatenate([v2, pad_block], axis=1).reshape(
        NC, NS, CHUNKS_PER_TILE, CHUNK)

  src = _pad_edges(ei[0])
  dst = _pad_edges(ei[1])

  degp = _deg_call()(src, dst)
  hs1 = _mm1(x, W1, degp)
  aggp1 = _agg_call()(hs1, src, dst)
  hs2 = _mid(aggp1, degp, b1, W2)
  aggp2 = _agg_call()(hs2, src, dst)
  return _final(aggp2, degp, b2)


# speedup vs baseline: 1.0897x; 1.0381x over previous
import functools

import jax
import jax.numpy as jnp
from jax import lax
from jax.experimental import pallas as pl
from jax.experimental.pallas import tpu as pltpu
from jax.experimental.pallas import tpu_sc as plsc

N_NODES = 10000
N_EDGES = 320000
NFEAT = 128

NC = 2
NS = 16
LANES = 16

CHUNK = 128
N_PAD = 10240
CHUNKS_PER_TILE = 80
EDGES_PER_TILE = CHUNKS_PER_TILE * CHUNK
E_PAD = EDGES_PER_TILE * NC * NS
ROWS_PER_TILE = N_PAD // NS
NBUF = 2

@functools.lru_cache(maxsize=None)
def _mesh():
  return plsc.VectorSubcoreMesh(
      core_axis_name="c", subcore_axis_name="s", num_cores=NC, num_subcores=NS
  )


def _zero_vmem_2d(ref, n_rows, n_cols):
  z = jnp.zeros((LANES,), jnp.float32)

  def body(i, _):
    for j in range(n_cols // LANES):
      ref[i, pl.ds(j * LANES, LANES)] = z
    return 0

  lax.fori_loop(0, n_rows, body, 0)


def _zero_vmem_1d(ref, n):
  z = jnp.zeros((LANES,), jnp.float32)
  for j in range(n // LANES):
    ref[pl.ds(j * LANES, LANES)] = z


def _deg_body(edges_hbm, out_hbm, sidx_v, didx_v, ones_v, zbuf_v,
              sem_a, sem_b, acc_out, acc_in):
  cid = lax.axis_index("c")
  sid = lax.axis_index("s")

  for j in range(CHUNK // LANES):
    ones_v[pl.ds(j * LANES, LANES)] = jnp.ones((LANES,), jnp.float32)
  _zero_vmem_1d(zbuf_v, ROWS_PER_TILE)

  pltpu.sync_copy(zbuf_v, acc_out.at[pl.ds(sid * ROWS_PER_TILE,
                                           ROWS_PER_TILE)])
  pltpu.sync_copy(zbuf_v, acc_in.at[pl.ds(sid * ROWS_PER_TILE,
                                          ROWS_PER_TILE)])
  pltpu.sync_copy(edges_hbm.at[0, cid, sid], sidx_v)
  pltpu.sync_copy(edges_hbm.at[1, cid, sid], didx_v)
  plsc.subcore_barrier()

  def chunk_body(k, _):
    pltpu.async_copy(ones_v, acc_out.at[sidx_v.at[k]], sem_a, add=True)
    pltpu.async_copy(ones_v, acc_in.at[didx_v.at[k]], sem_b, add=True)
    return 0

  lax.fori_loop(0, CHUNKS_PER_TILE, chunk_body, 0)

  def drain_body(k, _):
    pltpu.make_async_copy(ones_v, acc_out.at[sidx_v.at[0]], sem_a).wait()
    pltpu.make_async_copy(ones_v, acc_in.at[didx_v.at[0]], sem_b).wait()
    return 0

  lax.fori_loop(0, CHUNKS_PER_TILE, drain_body, 0)
  plsc.subcore_barrier()

  row = pl.ds(sid * ROWS_PER_TILE, ROWS_PER_TILE)
  pltpu.sync_copy(acc_out.at[row], out_hbm.at[cid, 0, row])
  pltpu.sync_copy(acc_in.at[row], out_hbm.at[cid, 1, row])


@functools.lru_cache(maxsize=None)
def _deg_call():
  return pl.kernel(
      _deg_body,
      out_type=jax.ShapeDtypeStruct((NC, 2, N_PAD), jnp.float32),
      mesh=_mesh(),
      scratch_types=[
          pltpu.VMEM((CHUNKS_PER_TILE, CHUNK), jnp.int32),
          pltpu.VMEM((CHUNKS_PER_TILE, CHUNK), jnp.int32),
          pltpu.VMEM((CHUNK,), jnp.float32),
          pltpu.VMEM((ROWS_PER_TILE,), jnp.float32),
          pltpu.SemaphoreType.DMA,
          pltpu.SemaphoreType.DMA,
          pltpu.VMEM_SHARED((N_PAD,), jnp.float32),
          pltpu.VMEM_SHARED((N_PAD,), jnp.float32),
      ],
  )


PHASES = 2
PHASE_CHUNKS = CHUNKS_PER_TILE // PHASES


def _agg_body(rows_hbm, edges_hbm, out_hbm, sidx_v, didx_v, bufs_v,
              semg0, semg1, acc):
  cid = lax.axis_index("c")
  sid = lax.axis_index("s")
  semg = (semg0, semg1)
  bufs = [bufs_v.at[b] for b in range(NBUF)]

  _zero_vmem_2d(bufs_v.at[0], CHUNK, NFEAT)
  for m in range(ROWS_PER_TILE // CHUNK):
    pltpu.sync_copy(
        bufs[0], acc.at[pl.ds(sid * ROWS_PER_TILE + m * CHUNK, CHUNK)])
  plsc.subcore_barrier()

  def gather(k, b):
    pltpu.async_copy(rows_hbm.at[sidx_v.at[k]], bufs[b], semg[b])

  def gather_wait(k, b):
    pltpu.make_async_copy(rows_hbm.at[sidx_v.at[k]], bufs[b],
                          semg[b]).wait()

  def scatter(k, b):
    pltpu.sync_copy(bufs[b], acc.at[didx_v.at[k]], add=True)

  for phase in range(PHASES):
    base = phase * PHASE_CHUNKS
    pltpu.sync_copy(edges_hbm.at[0, cid, sid, pl.ds(base, PHASE_CHUNKS)],
                    sidx_v)
    pltpu.sync_copy(edges_hbm.at[1, cid, sid, pl.ds(base, PHASE_CHUNKS)],
                    didx_v)

    for b in range(NBUF):
      gather(b, b)

    def round_body(i, _):
      for b in range(NBUF):
        k = i * NBUF + b
        gather_wait(k, b)
        scatter(k, b)
        gather(k + NBUF, b)
      return 0

    lax.fori_loop(0, PHASE_CHUNKS // NBUF - 1, round_body, 0)
    for b in range(NBUF):
      k = PHASE_CHUNKS - NBUF + b
      gather_wait(k, b)
      scatter(k, b)

  plsc.subcore_barrier()
  row = pl.ds(sid * ROWS_PER_TILE, ROWS_PER_TILE)
  pltpu.sync_copy(acc.at[row], out_hbm.at[cid, row])


@functools.lru_cache(maxsize=None)
def _agg_call():
  return pl.kernel(
      _agg_body,
      out_type=jax.ShapeDtypeStruct((NC, N_PAD, NFEAT), jnp.float32),
      mesh=_mesh(),
      scratch_types=[
          pltpu.VMEM((PHASE_CHUNKS, CHUNK), jnp.int32),
          pltpu.VMEM((PHASE_CHUNKS, CHUNK), jnp.int32),
          pltpu.VMEM((NBUF, CHUNK, NFEAT), jnp.float32),
          pltpu.SemaphoreType.DMA,
          pltpu.SemaphoreType.DMA,
          pltpu.VMEM_SHARED((N_PAD, NFEAT), jnp.float32),
      ],
  )


ROW_BLK = 1280
GRID = N_PAD // ROW_BLK


def _norm_from(deg_blk):
  return lax.rsqrt(jnp.maximum(deg_blk, 1.0))


def _mm1_body(x_ref, w_ref, degp_ref, o_ref):
  deg_out = degp_ref[0, 0, :] + degp_ref[1, 0, :]
  xs = x_ref[...] * _norm_from(deg_out)[:, None]
  o_ref[...] = jnp.dot(xs, w_ref[...], preferred_element_type=jnp.float32)


def _mm1(x, w, degp):
  return pl.pallas_call(
      _mm1_body,
      grid=(GRID,),
      in_specs=[
          pl.BlockSpec((ROW_BLK, NFEAT), lambda i: (i, 0)),
          pl.BlockSpec((NFEAT, NFEAT), lambda i: (0, 0)),
          pl.BlockSpec((NC, 2, ROW_BLK), lambda i: (0, 0, i)),
      ],
      out_specs=pl.BlockSpec((ROW_BLK, NFEAT), lambda i: (i, 0)),
      out_shape=jax.ShapeDtypeStruct((N_PAD, NFEAT), jnp.float32),
  )(x, w, degp)


def _mid_body(aggp_ref, degp_ref, b_ref, w_ref, o_ref):
  agg = aggp_ref[0] + aggp_ref[1]
  norm_dst = _norm_from(degp_ref[0, 1, :] + degp_ref[1, 1, :])
  norm_src = _norm_from(degp_ref[0, 0, :] + degp_ref[1, 0, :])
  h = jax.nn.relu(agg * norm_dst[:, None] + b_ref[...][None, :])
  hs = h * norm_src[:, None]
  o_ref[...] = jnp.dot(hs, w_ref[...], preferred_element_type=jnp.float32)


def _mid(aggp, degp, b, w):
  return pl.pallas_call(
      _mid_body,
      grid=(GRID,),
      in_specs=[
          pl.BlockSpec((NC, ROW_BLK, NFEAT), lambda i: (0, i, 0)),
          pl.BlockSpec((NC, 2, ROW_BLK), lambda i: (0, 0, i)),
          pl.BlockSpec((NFEAT,), lambda i: (0,)),
          pl.BlockSpec((NFEAT, NFEAT), lambda i: (0, 0)),
      ],
      out_specs=pl.BlockSpec((ROW_BLK, NFEAT), lambda i: (i, 0)),
      out_shape=jax.ShapeDtypeStruct((N_PAD, NFEAT), jnp.float32),
  )(aggp, degp, b, w)


def _final_body(aggp_ref, degp_ref, b_ref, o_ref):
  agg = aggp_ref[0] + aggp_ref[1]
  norm_dst = _norm_from(degp_ref[0, 1, :] + degp_ref[1, 1, :])
  o_ref[...] = agg * norm_dst[:, None] + b_ref[...][None, :]


def _final(aggp, degp, b):
  return pl.pallas_call(
      _final_body,
      grid=(GRID,),
      in_specs=[
          pl.BlockSpec((NC, ROW_BLK, NFEAT), lambda i: (0, i, 0)),
          pl.BlockSpec((NC, 2, ROW_BLK), lambda i: (0, 0, i)),
          pl.BlockSpec((NFEAT,), lambda i: (0,)),
      ],
      out_specs=pl.BlockSpec((ROW_BLK, NFEAT), lambda i: (i, 0)),
      out_shape=jax.ShapeDtypeStruct((N_NODES, NFEAT), jnp.float32),
  )(aggp, degp, b)


def kernel(x, W1, b1, W2, b2, edge_index):
  ei = edge_index.astype(jnp.int32)
  real_per_tile = N_EDGES // (NC * NS)
  pad_per_tile = EDGES_PER_TILE - real_per_tile
  pad_block = jnp.broadcast_to(
      (N_NODES + jnp.arange(pad_per_tile, dtype=jnp.int32))[None, None, :],
      (2, NC * NS, pad_per_tile))
  edges = jnp.concatenate(
      [ei.reshape(2, NC * NS, real_per_tile), pad_block], axis=2
  ).reshape(2, NC, NS, CHUNKS_PER_TILE, CHUNK)

  degp = _deg_call()(edges)
  hs1 = _mm1(x, W1, degp)
  aggp1 = _agg_call()(hs1, edges)
  hs2 = _mid(aggp1, degp, b1, W2)
  aggp2 = _agg_call()(hs2, edges)
  return _final(aggp2, degp, b2)


# speedup vs baseline: 1.0911x; 1.0013x over previous
import functools

import jax
import jax.numpy as jnp
from jax import lax
from jax.experimental import pallas as pl
from jax.experimental.pallas import tpu as pltpu
from jax.experimental.pallas import tpu_sc as plsc

N_NODES = 10000
N_EDGES = 320000
NFEAT = 128

NC = 2
NS = 16
LANES = 16

CHUNK = 128
N_PAD = 10240
CHUNKS_PER_TILE = 80
EDGES_PER_TILE = CHUNKS_PER_TILE * CHUNK
E_PAD = EDGES_PER_TILE * NC * NS
ROWS_PER_TILE = N_PAD // NS
NBUF = 2

@functools.lru_cache(maxsize=None)
def _mesh():
  return plsc.VectorSubcoreMesh(
      core_axis_name="c", subcore_axis_name="s", num_cores=NC, num_subcores=NS
  )


def _zero_vmem_2d(ref, n_rows, n_cols):
  z = jnp.zeros((LANES,), jnp.float32)

  def body(i, _):
    for j in range(n_cols // LANES):
      ref[i, pl.ds(j * LANES, LANES)] = z
    return 0

  lax.fori_loop(0, n_rows, body, 0)


def _zero_vmem_1d(ref, n):
  z = jnp.zeros((LANES,), jnp.float32)
  for j in range(n // LANES):
    ref[pl.ds(j * LANES, LANES)] = z


def _deg_body(edges_hbm, out_hbm, sidx_v, didx_v, ones_v, zbuf_v,
              sem_a, sem_b, acc_out, acc_in):
  cid = lax.axis_index("c")
  sid = lax.axis_index("s")

  for j in range(CHUNK // LANES):
    ones_v[pl.ds(j * LANES, LANES)] = jnp.ones((LANES,), jnp.float32)
  _zero_vmem_1d(zbuf_v, ROWS_PER_TILE)

  pltpu.sync_copy(zbuf_v, acc_out.at[pl.ds(sid * ROWS_PER_TILE,
                                           ROWS_PER_TILE)])
  pltpu.sync_copy(zbuf_v, acc_in.at[pl.ds(sid * ROWS_PER_TILE,
                                          ROWS_PER_TILE)])
  pltpu.sync_copy(edges_hbm.at[0, cid, sid], sidx_v)
  pltpu.sync_copy(edges_hbm.at[1, cid, sid], didx_v)
  plsc.subcore_barrier()

  def chunk_body(k, _):
    pltpu.async_copy(ones_v, acc_out.at[sidx_v.at[k]], sem_a, add=True)
    pltpu.async_copy(ones_v, acc_in.at[didx_v.at[k]], sem_b, add=True)
    return 0

  lax.fori_loop(0, CHUNKS_PER_TILE, chunk_body, 0)

  pltpu.make_async_copy(edges_hbm.at[0, cid, sid], sidx_v, sem_a).wait()
  pltpu.make_async_copy(edges_hbm.at[1, cid, sid], didx_v, sem_b).wait()
  plsc.subcore_barrier()

  row = pl.ds(sid * ROWS_PER_TILE, ROWS_PER_TILE)
  pltpu.sync_copy(acc_out.at[row], out_hbm.at[cid, 0, row])
  pltpu.sync_copy(acc_in.at[row], out_hbm.at[cid, 1, row])


@functools.lru_cache(maxsize=None)
def _deg_call():
  return pl.kernel(
      _deg_body,
      out_type=jax.ShapeDtypeStruct((NC, 2, N_PAD), jnp.float32),
      mesh=_mesh(),
      scratch_types=[
          pltpu.VMEM((CHUNKS_PER_TILE, CHUNK), jnp.int32),
          pltpu.VMEM((CHUNKS_PER_TILE, CHUNK), jnp.int32),
          pltpu.VMEM((CHUNK,), jnp.float32),
          pltpu.VMEM((ROWS_PER_TILE,), jnp.float32),
          pltpu.SemaphoreType.DMA,
          pltpu.SemaphoreType.DMA,
          pltpu.VMEM_SHARED((N_PAD,), jnp.float32),
          pltpu.VMEM_SHARED((N_PAD,), jnp.float32),
      ],
  )


PHASES = 2
PHASE_CHUNKS = CHUNKS_PER_TILE // PHASES


def _agg_body(rows_hbm, edges_hbm, out_hbm, sidx_v, didx_v, bufs_v,
              semg0, semg1, acc):
  cid = lax.axis_index("c")
  sid = lax.axis_index("s")
  semg = (semg0, semg1)
  bufs = [bufs_v.at[b] for b in range(NBUF)]

  _zero_vmem_2d(bufs_v.at[0], CHUNK, NFEAT)
  for m in range(ROWS_PER_TILE // CHUNK):
    pltpu.sync_copy(
        bufs[0], acc.at[pl.ds(sid * ROWS_PER_TILE + m * CHUNK, CHUNK)])
  plsc.subcore_barrier()

  def gather(k, b):
    pltpu.async_copy(rows_hbm.at[sidx_v.at[k]], bufs[b], semg[b])

  def gather_wait(k, b):
    pltpu.make_async_copy(rows_hbm.at[sidx_v.at[k]], bufs[b],
                          semg[b]).wait()

  def scatter(k, b):
    pltpu.sync_copy(bufs[b], acc.at[didx_v.at[k]], add=True)

  for phase in range(PHASES):
    base = phase * PHASE_CHUNKS
    pltpu.sync_copy(edges_hbm.at[0, cid, sid, pl.ds(base, PHASE_CHUNKS)],
                    sidx_v)
    pltpu.sync_copy(edges_hbm.at[1, cid, sid, pl.ds(base, PHASE_CHUNKS)],
                    didx_v)

    for b in range(NBUF):
      gather(b, b)

    def round_body(i, _):
      for b in range(NBUF):
        k = i * NBUF + b
        gather_wait(k, b)
        scatter(k, b)
        gather(k + NBUF, b)
      return 0

    lax.fori_loop(0, PHASE_CHUNKS // NBUF - 1, round_body, 0)
    for b in range(NBUF):
      k = PHASE_CHUNKS - NBUF + b
      gather_wait(k, b)
      scatter(k, b)

  plsc.subcore_barrier()
  row = pl.ds(sid * ROWS_PER_TILE, ROWS_PER_TILE)
  pltpu.sync_copy(acc.at[row], out_hbm.at[cid, row])


@functools.lru_cache(maxsize=None)
def _agg_call():
  return pl.kernel(
      _agg_body,
      out_type=jax.ShapeDtypeStruct((NC, N_PAD, NFEAT), jnp.float32),
      mesh=_mesh(),
      scratch_types=[
          pltpu.VMEM((PHASE_CHUNKS, CHUNK), jnp.int32),
          pltpu.VMEM((PHASE_CHUNKS, CHUNK), jnp.int32),
          pltpu.VMEM((NBUF, CHUNK, NFEAT), jnp.float32),
          pltpu.SemaphoreType.DMA,
          pltpu.SemaphoreType.DMA,
          pltpu.VMEM_SHARED((N_PAD, NFEAT), jnp.float32),
      ],
  )


ROW_BLK = 1280
GRID = N_PAD // ROW_BLK


def _norm_from(deg_blk):
  return lax.rsqrt(jnp.maximum(deg_blk, 1.0))


def _mm1_body(x_ref, w_ref, degp_ref, o_ref):
  deg_out = degp_ref[0, 0, :] + degp_ref[1, 0, :]
  xs = x_ref[...] * _norm_from(deg_out)[:, None]
  o_ref[...] = jnp.dot(xs, w_ref[...], preferred_element_type=jnp.float32)


def _mm1(x, w, degp):
  return pl.pallas_call(
      _mm1_body,
      grid=(GRID,),
      in_specs=[
          pl.BlockSpec((ROW_BLK, NFEAT), lambda i: (i, 0)),
          pl.BlockSpec((NFEAT, NFEAT), lambda i: (0, 0)),
          pl.BlockSpec((NC, 2, ROW_BLK), lambda i: (0, 0, i)),
      ],
      out_specs=pl.BlockSpec((ROW_BLK, NFEAT), lambda i: (i, 0)),
      out_shape=jax.ShapeDtypeStruct((N_PAD, NFEAT), jnp.float32),
  )(x, w, degp)


def _mid_body(aggp_ref, degp_ref, b_ref, w_ref, o_ref):
  agg = aggp_ref[0] + aggp_ref[1]
  norm_dst = _norm_from(degp_ref[0, 1, :] + degp_ref[1, 1, :])
  norm_src = _norm_from(degp_ref[0, 0, :] + degp_ref[1, 0, :])
  h = jax.nn.relu(agg * norm_dst[:, None] + b_ref[...][None, :])
  hs = h * norm_src[:, None]
  o_ref[...] = jnp.dot(hs, w_ref[...], preferred_element_type=jnp.float32)


def _mid(aggp, degp, b, w):
  return pl.pallas_call(
      _mid_body,
      grid=(GRID,),
      in_specs=[
          pl.BlockSpec((NC, ROW_BLK, NFEAT), lambda i: (0, i, 0)),
          pl.BlockSpec((NC, 2, ROW_BLK), lambda i: (0, 0, i)),
          pl.BlockSpec((NFEAT,), lambda i: (0,)),
          pl.BlockSpec((NFEAT, NFEAT), lambda i: (0, 0)),
      ],
      out_specs=pl.BlockSpec((ROW_BLK, NFEAT), lambda i: (i, 0)),
      out_shape=jax.ShapeDtypeStruct((N_PAD, NFEAT), jnp.float32),
  )(aggp, degp, b, w)


def _final_body(aggp_ref, degp_ref, b_ref, o_ref):
  agg = aggp_ref[0] + aggp_ref[1]
  norm_dst = _norm_from(degp_ref[0, 1, :] + degp_ref[1, 1, :])
  o_ref[...] = agg * norm_dst[:, None] + b_ref[...][None, :]


def _final(aggp, degp, b):
  return pl.pallas_call(
      _final_body,
      grid=(GRID,),
      in_specs=[
          pl.BlockSpec((NC, ROW_BLK, NFEAT), lambda i: (0, i, 0)),
          pl.BlockSpec((NC, 2, ROW_BLK), lambda i: (0, 0, i)),
          pl.BlockSpec((NFEAT,), lambda i: (0,)),
      ],
      out_specs=pl.BlockSpec((ROW_BLK, NFEAT), lambda i: (i, 0)),
      out_shape=jax.ShapeDtypeStruct((N_NODES, NFEAT), jnp.float32),
  )(aggp, degp, b)


def kernel(x, W1, b1, W2, b2, edge_index):
  ei = edge_index.astype(jnp.int32)
  real_per_tile = N_EDGES // (NC * NS)
  pad_per_tile = EDGES_PER_TILE - real_per_tile
  pad_block = jnp.broadcast_to(
      (N_NODES + jnp.arange(pad_per_tile, dtype=jnp.int32))[None, None, :],
      (2, NC * NS, pad_per_tile))
  edges = jnp.concatenate(
      [ei.reshape(2, NC * NS, real_per_tile), pad_block], axis=2
  ).reshape(2, NC, NS, CHUNKS_PER_TILE, CHUNK)

  degp = _deg_call()(edges)
  hs1 = _mm1(x, W1, degp)
  aggp1 = _agg_call()(hs1, edges)
  hs2 = _mid(aggp1, degp, b1, W2)
  aggp2 = _agg_call()(hs2, edges)
  return _final(aggp2, degp, b2)
